# baseline (speedup 1.0000x reference)
import jax
import jax.numpy as jnp
from jax import lax
from jax.experimental import pallas as pl
from jax.experimental.pallas import tpu as pltpu
from jax.experimental.pallas import tpu_sc as plsc

_SRC_X = 2048
_SRC_Y = 2048
_ROWS = 4096
_COLS = 4096
_NC = 2
_NS = 16
_NW = _NC * _NS
_RPW = _ROWS // _NW
_CH = 4
_NBUF = 7
_LAG = 3
_NCHUNK = _RPW // _CH

_SRC_W = _SRC_X // _RPW
_SRC_CHUNK = (_SRC_X % _RPW) // _CH
_SRC_R = _SRC_X % _CH
_SRC_LANE_BASE = (_SRC_Y // 16) * 16
_SRC_LANE = _SRC_Y % 16


def _sc_body(b_hbm, bt_hbm, o_hbm, *scratch):
    bufs = scratch[:_NBUF]
    btv = scratch[_NBUF]
    sin = scratch[_NBUF + 1 : _NBUF + 1 + _NBUF]
    sout = scratch[_NBUF + 1 + _NBUF :]

    wid = lax.axis_index("s") * _NC + lax.axis_index("c")
    row0 = wid * _RPW

    def in_cp(g):
        return pltpu.make_async_copy(
            b_hbm.at[pl.ds(row0 + g * _CH, _CH), :], bufs[g % _NBUF], sin[g % _NBUF]
        )

    def out_cp(g):
        return pltpu.make_async_copy(
            bufs[g % _NBUF], o_hbm.at[pl.ds(row0 + g * _CH, _CH), :], sout[g % _NBUF]
        )

    pltpu.sync_copy(bt_hbm, btv.at[pl.ds(0, 1)])

    for g in range(_NBUF):
        in_cp(g).start()

    for g in range(_NCHUNK):
        in_cp(g).wait()
        if g == _SRC_CHUNK:
            @pl.when(wid == _SRC_W)
            def _():
                buf = bufs[g % _NBUF]
                cur = buf[_SRC_R, pl.ds(_SRC_LANE_BASE, 16)]
                lane = lax.iota(jnp.int32, 16)
                buf[_SRC_R, pl.ds(_SRC_LANE_BASE, 16)] = jnp.where(
                    lane == _SRC_LANE, btv[...], cur
                )
        out_cp(g).start()
        h = g - _LAG
        if 0 <= h and h + _NBUF < _NCHUNK:
            out_cp(h).wait()
            in_cp(h + _NBUF).start()

    for g in range(_NCHUNK - _NBUF, _NCHUNK):
        out_cp(g).wait()


def kernel(B, Bt):
    mesh = plsc.VectorSubcoreMesh(core_axis_name="c", subcore_axis_name="s")
    scratch = (
        [pltpu.VMEM((_CH, _COLS), jnp.float32) for _ in range(_NBUF)]
        + [pltpu.VMEM((16,), jnp.float32)]
        + [pltpu.SemaphoreType.DMA for _ in range(2 * _NBUF)]
    )
    f = pl.kernel(
        _sc_body,
        out_type=jax.ShapeDtypeStruct((_ROWS, _COLS), jnp.float32),
        mesh=mesh,
        scratch_types=scratch,
    )
    out = f(B.reshape(_ROWS, _COLS), Bt.reshape(1))
    return out.reshape(1, _ROWS, _COLS)


# speedup vs baseline: 1.6124x; 1.6124x over previous
import jax
import jax.numpy as jnp
from jax.experimental import pallas as pl
from jax.experimental.pallas import tpu as pltpu

_SRC_X = 2048
_SRC_Y = 2048
_ROWS = 4096
_COLS = 4096
_CHR = 256
_NBUF = 16
_NCHUNK = _ROWS // _CHR
_SRC_CHUNK = _SRC_X // _CHR
_SRC_R = _SRC_X % _CHR


def _body(b_hbm, bt_smem, o_hbm, *scratch):
    bufs = scratch[:_NBUF]
    sin = scratch[_NBUF : 2 * _NBUF]
    sout = scratch[2 * _NBUF :]

    def in_cp(g):
        return pltpu.make_async_copy(
            b_hbm.at[:, pl.ds(g * _CHR, _CHR), :], bufs[g % _NBUF], sin[g % _NBUF]
        )

    def out_cp(g):
        return pltpu.make_async_copy(
            bufs[g % _NBUF], o_hbm.at[:, pl.ds(g * _CHR, _CHR), :], sout[g % _NBUF]
        )

    for g in range(_NBUF):
        in_cp(g).start()

    for g in range(_NCHUNK):
        in_cp(g).wait()
        if g == _SRC_CHUNK:
            buf = bufs[g % _NBUF]
            col_ids = jax.lax.broadcasted_iota(jnp.int32, (1, _COLS), 1)
            buf[0, _SRC_R : _SRC_R + 1, :] = jnp.where(
                col_ids == _SRC_Y, bt_smem[0, 0], buf[0, _SRC_R : _SRC_R + 1, :]
            )
        out_cp(g).start()
        if g + _NBUF < _NCHUNK:
            out_cp(g).wait()
            in_cp(g + _NBUF).start()

    for g in range(_NCHUNK - _NBUF, _NCHUNK):
        out_cp(g).wait()


def kernel(B, Bt):
    scratch = (
        [pltpu.VMEM((1, _CHR, _COLS), jnp.float32) for _ in range(_NBUF)]
        + [pltpu.SemaphoreType.DMA for _ in range(2 * _NBUF)]
    )
    return pl.pallas_call(
        _body,
        in_specs=[
            pl.BlockSpec(memory_space=pl.ANY),
            pl.BlockSpec(memory_space=pltpu.SMEM),
        ],
        out_specs=pl.BlockSpec(memory_space=pl.ANY),
        out_shape=jax.ShapeDtypeStruct((1, _ROWS, _COLS), jnp.float32),
        scratch_shapes=scratch,
    )(B, Bt)
